# baseline (speedup 1.0000x reference)
import functools

import jax
import jax.numpy as jnp
from jax import lax
from jax.experimental import pallas as pl
from jax.experimental.pallas import tpu as pltpu
from jax.experimental.pallas import tpu_sc as plsc

TOKENS_PER_BLOCK = 64
TOKENS_PER_SUB_CHUNK = 16
SUBS_PER_BLOCK = TOKENS_PER_BLOCK // TOKENS_PER_SUB_CHUNK
NUM_PAGES = 512

_CHUNK = 4096


def _pool_body(x_ref, o_ref):
    x = x_ref[...]
    n, h, d = x.shape
    xr = x.reshape(n // TOKENS_PER_SUB_CHUNK, TOKENS_PER_SUB_CHUNK, h, d)
    o_ref[0] = jnp.min(xr, axis=1)
    o_ref[1] = jnp.max(xr, axis=1)


def _pool(keys, T, H, D):
    n_sub = T // TOKENS_PER_SUB_CHUNK
    return pl.pallas_call(
        _pool_body,
        grid=(T // _CHUNK,),
        in_specs=[pl.BlockSpec((_CHUNK, H, D), lambda i: (i, 0, 0))],
        out_specs=pl.BlockSpec(
            (2, _CHUNK // TOKENS_PER_SUB_CHUNK, H, D), lambda i: (0, i, 0, 0)
        ),
        out_shape=jax.ShapeDtypeStruct((2, n_sub, H, D), jnp.float32),
    )(keys)


def _make_sc_scatter(n_blocks, n_heads, n_pool, n_seq, max_blocks_per_seq):
    mesh = plsc.VectorSubcoreMesh(core_axis_name="c", subcore_axis_name="s")
    blocks_per_sub = n_blocks // 16
    rows_per_block = SUBS_PER_BLOCK * n_pool
    half_src = n_blocks * SUBS_PER_BLOCK * n_heads
    half_dst = NUM_PAGES * SUBS_PER_BLOCK * n_pool
    out_rows = 2 * half_dst
    n_idx = blocks_per_sub * rows_per_block
    zrows = 128

    @functools.partial(
        pl.kernel,
        mesh=mesh,
        out_type=jax.ShapeDtypeStruct((out_rows, 128), jnp.float32),
        scratch_types=[
            pltpu.VMEM((NUM_PAGES + 32,), jnp.int32),
            pltpu.VMEM((128,), jnp.int32),
            pltpu.VMEM((128,), jnp.int32),
            pltpu.VMEM((128,), jnp.int32),
            pltpu.VMEM((128,), jnp.int32),
            pltpu.VMEM((n_idx, 128), jnp.float32),
            pltpu.VMEM((zrows, 128), jnp.float32),
            pltpu.SemaphoreType.DMA,
        ],
        compiler_params=pltpu.CompilerParams(needs_layout_passes=False),
    )
    def sc_scatter(pooled_hbm, tbl_hbm, zeros_hbm, out_hbm,
                   tbl_v, idx_src_a, idx_src_b,
                   idx_dst_a, idx_dst_b, stage_v, zero_v, sem):
        c = lax.axis_index("c")
        s = lax.axis_index("s")
        cu_off = NUM_PAGES
        hd_off = NUM_PAGES + 17
        ld = [pltpu.async_copy(tbl_hbm, tbl_v, sem),
              pltpu.async_copy(zeros_hbm, zero_v, sem)]
        for h in ld:
            h.wait()
        rows_per_sub = half_dst // 16
        base = c * half_dst + s * rows_per_sub
        zfill = [
            pltpu.async_copy(zero_v, out_hbm.at[pl.ds(base + r, zrows)], sem)
            for r in range(0, rows_per_sub, zrows)
        ]
        iota = lax.iota(jnp.int32, 16)
        b_vec = s * blocks_per_sub + iota
        t_vec = b_vec * TOKENS_PER_BLOCK
        seq = jnp.zeros((16,), jnp.int32)
        for j in range(1, n_seq + 1):
            cj = plsc.load_gather(
                tbl_v, [jnp.full((16,), cu_off + j, jnp.int32)])
            seq = seq + (cj <= t_vec).astype(jnp.int32)
        cu_s = plsc.load_gather(tbl_v, [seq + cu_off])
        flat = seq * max_blocks_per_seq + (t_vec - cu_s) // TOKENS_PER_BLOCK
        pages = plsc.load_gather(tbl_v, [flat])
        src_base = (c * half_src
                    + (s * blocks_per_sub + iota) * (SUBS_PER_BLOCK * n_heads))
        dst_base = c * half_dst + pages * rows_per_block
        for k in range(rows_per_block):
            sub, h_slot = k // n_pool, k % n_pool
            head_val = plsc.load_gather(
                tbl_v, [jnp.full((16,), hd_off + h_slot, jnp.int32)])
            dst = dst_base + (sub * n_pool + h_slot)
            src = src_base + sub * n_heads + head_val
            dref = idx_dst_a if k < 8 else idx_dst_b
            sref = idx_src_a if k < 8 else idx_src_b
            dref[pl.ds((k & 7) * 16, 16)] = dst
            sref[pl.ds((k & 7) * 16, 16)] = src
        g0 = pltpu.async_copy(pooled_hbm.at[idx_src_a],
                              stage_v.at[pl.ds(0, 128)], sem)
        g1 = pltpu.async_copy(pooled_hbm.at[idx_src_b],
                              stage_v.at[pl.ds(128, 128)], sem)
        for h in zfill:
            h.wait()
        g0.wait()
        g1.wait()
        plsc.subcore_barrier()
        s0 = pltpu.async_copy(stage_v.at[pl.ds(0, 128)],
                              out_hbm.at[idx_dst_a], sem)
        s1 = pltpu.async_copy(stage_v.at[pl.ds(128, 128)],
                              out_hbm.at[idx_dst_b], sem)
        s0.wait()
        s1.wait()

    return sc_scatter


def kernel(keys, block_tables, cu_seqlens, pooling_heads_idx,
           num_retrieval_kv_heads):
    del num_retrieval_kv_heads
    T, H, D = keys.shape
    P = pooling_heads_idx.shape[0]
    n_seq = cu_seqlens.shape[0] - 1
    n_blocks = T // TOKENS_PER_BLOCK

    pooled = _pool(keys, T, H, D)
    pooled_rows = pooled.reshape(2 * (T // TOKENS_PER_SUB_CHUNK) * H, D)

    n_cu = cu_seqlens.shape[0]
    tbl = jnp.concatenate([
        block_tables.reshape(-1).astype(jnp.int32),
        cu_seqlens.astype(jnp.int32),
        jnp.full((17 - n_cu,), 0x3FFFFFFF, jnp.int32),
        pooling_heads_idx.astype(jnp.int32),
        jnp.zeros((32 - 17 - P,), jnp.int32),
    ])
    zeros = jnp.zeros((128, D), jnp.float32)

    scatter = _make_sc_scatter(n_blocks, H, P, n_seq, block_tables.shape[1])
    out = scatter(pooled_rows, tbl, zeros)
    return out.reshape(2, NUM_PAGES * SUBS_PER_BLOCK, P, D)


# speedup vs baseline: 1.0348x; 1.0348x over previous
import functools

import jax
import jax.numpy as jnp
from jax import lax
from jax.experimental import pallas as pl
from jax.experimental.pallas import tpu as pltpu
from jax.experimental.pallas import tpu_sc as plsc

TOKENS_PER_BLOCK = 64
TOKENS_PER_SUB_CHUNK = 16
SUBS_PER_BLOCK = TOKENS_PER_BLOCK // TOKENS_PER_SUB_CHUNK
NUM_PAGES = 512

_CHUNK = 2048


def _pool_body(x_ref, o_ref):
    x = x_ref[...]
    n, h, d = x.shape
    xr = x.reshape(n // TOKENS_PER_SUB_CHUNK, TOKENS_PER_SUB_CHUNK, h, d)
    o_ref[0] = jnp.min(xr, axis=1)
    o_ref[1] = jnp.max(xr, axis=1)


def _pool(keys, T, H, D):
    n_sub = T // TOKENS_PER_SUB_CHUNK
    return pl.pallas_call(
        _pool_body,
        grid=(T // _CHUNK,),
        in_specs=[pl.BlockSpec((_CHUNK, H, D), lambda i: (i, 0, 0))],
        out_specs=pl.BlockSpec(
            (2, _CHUNK // TOKENS_PER_SUB_CHUNK, H, D), lambda i: (0, i, 0, 0)
        ),
        out_shape=jax.ShapeDtypeStruct((2, n_sub, H, D), jnp.float32),
    )(keys)


def _make_sc_scatter(n_blocks, n_heads, n_pool, n_seq, max_blocks_per_seq):
    mesh = plsc.VectorSubcoreMesh(core_axis_name="c", subcore_axis_name="s")
    blocks_per_sub = n_blocks // 16
    rows_per_block = SUBS_PER_BLOCK * n_pool
    half_src = n_blocks * SUBS_PER_BLOCK * n_heads
    half_dst = NUM_PAGES * SUBS_PER_BLOCK * n_pool
    out_rows = 2 * half_dst
    n_idx = blocks_per_sub * rows_per_block
    zrows = 128

    @functools.partial(
        pl.kernel,
        mesh=mesh,
        out_type=jax.ShapeDtypeStruct((out_rows, 128), jnp.float32),
        scratch_types=[
            pltpu.VMEM((NUM_PAGES + 32,), jnp.int32),
            pltpu.VMEM((128,), jnp.int32),
            pltpu.VMEM((128,), jnp.int32),
            pltpu.VMEM((128,), jnp.int32),
            pltpu.VMEM((128,), jnp.int32),
            pltpu.VMEM((128,), jnp.int32),
            pltpu.VMEM((128,), jnp.int32),
            pltpu.VMEM((128,), jnp.int32),
            pltpu.VMEM((128,), jnp.int32),
            pltpu.VMEM((n_idx, 128), jnp.float32),
            pltpu.VMEM((zrows, 128), jnp.float32),
            pltpu.SemaphoreType.DMA,
        ],
        compiler_params=pltpu.CompilerParams(needs_layout_passes=False),
    )
    def sc_scatter(pooled_hbm, tbl_hbm, zeros_hbm, out_hbm,
                   tbl_v, idx_src_a, idx_src_b, idx_dst_a, idx_dst_b,
                   idx_z0, idx_z1, idx_z2, idx_z3, stage_v, zero_v, sem):
        c = lax.axis_index("c")
        s = lax.axis_index("s")
        cu_off = NUM_PAGES
        hd_off = NUM_PAGES + 17
        ld = [pltpu.async_copy(tbl_hbm, tbl_v, sem),
              pltpu.async_copy(zeros_hbm, zero_v, sem)]
        for h in ld:
            h.wait()
        iota = lax.iota(jnp.int32, 16)
        b_vec = s * blocks_per_sub + iota
        t_vec = b_vec * TOKENS_PER_BLOCK
        seq = jnp.zeros((16,), jnp.int32)
        for j in range(1, n_seq + 1):
            cj = plsc.load_gather(
                tbl_v, [jnp.full((16,), cu_off + j, jnp.int32)])
            seq = seq + (cj <= t_vec).astype(jnp.int32)
        cu_s = plsc.load_gather(tbl_v, [seq + cu_off])
        flat = seq * max_blocks_per_seq + (t_vec - cu_s) // TOKENS_PER_BLOCK
        pages = plsc.load_gather(tbl_v, [flat])
        src_base = (c * half_src
                    + (s * blocks_per_sub + iota) * (SUBS_PER_BLOCK * n_heads))
        dst_base = c * half_dst + pages * rows_per_block
        zsc = []
        for ch, zrefs in ((0, (idx_z0, idx_z1)), (1, (idx_z2, idx_z3))):
            e = s + 16 * (iota + 16 * ch)
            seqv = e // max_blocks_per_seq
            blkv = e % max_blocks_per_seq
            cu_lo = plsc.load_gather(tbl_v, [seqv + cu_off])
            cu_hi = plsc.load_gather(tbl_v, [seqv + 1 + cu_off])
            nblk = (cu_hi - cu_lo) // TOKENS_PER_BLOCK
            used_e = blkv < nblk
            page_e = plsc.load_gather(tbl_v, [e])
            zbase = jnp.where(used_e, dst_base,
                              c * half_dst + page_e * rows_per_block)
            for k in range(rows_per_block):
                zrefs[k // 8][pl.ds((k & 7) * 16, 16)] = zbase + k
            zsc.append(pltpu.async_copy(zero_v, out_hbm.at[zrefs[0]], sem))
            zsc.append(pltpu.async_copy(zero_v, out_hbm.at[zrefs[1]], sem))
        for k in range(rows_per_block):
            sub, h_slot = k // n_pool, k % n_pool
            head_val = plsc.load_gather(
                tbl_v, [jnp.full((16,), hd_off + h_slot, jnp.int32)])
            dst = dst_base + (sub * n_pool + h_slot)
            src = src_base + sub * n_heads + head_val
            dref = idx_dst_a if k < 8 else idx_dst_b
            sref = idx_src_a if k < 8 else idx_src_b
            dref[pl.ds((k & 7) * 16, 16)] = dst
            sref[pl.ds((k & 7) * 16, 16)] = src
        g0 = pltpu.async_copy(pooled_hbm.at[idx_src_a],
                              stage_v.at[pl.ds(0, 128)], sem)
        g1 = pltpu.async_copy(pooled_hbm.at[idx_src_b],
                              stage_v.at[pl.ds(128, 128)], sem)
        for h in zsc:
            h.wait()
        g0.wait()
        g1.wait()
        s0 = pltpu.async_copy(stage_v.at[pl.ds(0, 128)],
                              out_hbm.at[idx_dst_a], sem)
        s1 = pltpu.async_copy(stage_v.at[pl.ds(128, 128)],
                              out_hbm.at[idx_dst_b], sem)
        s0.wait()
        s1.wait()

    return sc_scatter


def kernel(keys, block_tables, cu_seqlens, pooling_heads_idx,
           num_retrieval_kv_heads):
    del num_retrieval_kv_heads
    T, H, D = keys.shape
    P = pooling_heads_idx.shape[0]
    n_seq = cu_seqlens.shape[0] - 1
    n_blocks = T // TOKENS_PER_BLOCK

    pooled = _pool(keys, T, H, D)
    pooled_rows = pooled.reshape(2 * (T // TOKENS_PER_SUB_CHUNK) * H, D)

    n_cu = cu_seqlens.shape[0]
    tbl = jnp.concatenate([
        block_tables.reshape(-1).astype(jnp.int32),
        cu_seqlens.astype(jnp.int32),
        jnp.full((17 - n_cu,), 0x3FFFFFFF, jnp.int32),
        pooling_heads_idx.astype(jnp.int32),
        jnp.zeros((32 - 17 - P,), jnp.int32),
    ])
    zeros = jnp.zeros((128, D), jnp.float32)

    scatter = _make_sc_scatter(n_blocks, H, P, n_seq, block_tables.shape[1])
    out = scatter(pooled_rows, tbl, zeros)
    return out.reshape(2, NUM_PAGES * SUBS_PER_BLOCK, P, D)
